# Initial kernel scaffold; baseline (speedup 1.0000x reference)
import jax
import jax.numpy as jnp
from jax.experimental import pallas as pl


def kernel(loc, deadline, depot, W0_w, W0_b, W2_w, W2_b):
    raise NotImplementedError("write your pallas kernel here")


# speedup vs baseline: 1.7883x; 1.7883x over previous
import functools

import jax
import jax.numpy as jnp
from jax.experimental import pallas as pl
from jax.experimental.pallas import tpu as pltpu

_THRESH = 0.055
_N = 500
_E = 128


def _ccn2_body(feat_ref, featT_ref, w0t_ref, w0b_ref, w2t_ref, w2b_ref,
               out_ref, mean_ref):
    f = feat_ref[0]
    ft = featT_ref[0]
    xc = f[:, 0:1]
    yc = f[:, 1:2]
    xr = ft[0:1, :]
    yr = ft[1:2, :]
    dx = xc - xr
    dy = yc - yr
    dist2 = dx * dx + dy * dy
    A = (dist2 <= _THRESH * _THRESH).astype(jnp.float32)
    Ab = A.astype(jnp.bfloat16)

    fv0 = jnp.maximum(
        jnp.dot(f, w0t_ref[...], preferred_element_type=jnp.float32)
        + w0b_ref[...], 0.0)
    fv1 = jnp.dot(A, fv0, preferred_element_type=jnp.float32)

    C = jnp.dot(Ab, Ab, preferred_element_type=jnp.float32)
    B2 = (C > 0).astype(jnp.float32)
    D = jnp.dot(B2.astype(jnp.bfloat16), Ab, preferred_element_type=jnp.float32)
    M = B2 * D

    fv2 = jnp.dot(M, fv1, preferred_element_type=jnp.float32)
    Fv2 = jnp.maximum(
        jnp.dot(fv2, w2t_ref[...], preferred_element_type=jnp.float32)
        + w2b_ref[...], 0.0)
    out_ref[0] = Fv2
    mean_ref[0, 0] = jnp.mean(Fv2, axis=0)


@functools.partial(jax.jit, static_argnames=())
def kernel(loc, deadline, depot, W0_w, W0_b, W2_w, W2_b):
    B = loc.shape[0]
    locations = jnp.concatenate([depot[:, None, :], loc], axis=1)
    td = jnp.concatenate(
        [jnp.zeros((B, 1), deadline.dtype), deadline], axis=1)
    feat = jnp.concatenate([locations, td[..., None]], axis=-1)
    featT = jnp.swapaxes(feat, 1, 2)
    w0t = W0_w.T
    w2t = W2_w.T
    w0b = W0_b[None, :]
    w2b = W2_b[None, :]

    grid = (B,)
    out_shape = (
        jax.ShapeDtypeStruct((B, _N, _E), jnp.float32),
        jax.ShapeDtypeStruct((B, 1, _E), jnp.float32),
    )
    Fv2, mean = pl.pallas_call(
        _ccn2_body,
        grid=grid,
        in_specs=[
            pl.BlockSpec((1, _N, 3), lambda b: (b, 0, 0)),
            pl.BlockSpec((1, 3, _N), lambda b: (b, 0, 0)),
            pl.BlockSpec((3, _E), lambda b: (0, 0)),
            pl.BlockSpec((1, _E), lambda b: (0, 0)),
            pl.BlockSpec((_E, _E), lambda b: (0, 0)),
            pl.BlockSpec((1, _E), lambda b: (0, 0)),
        ],
        out_specs=(
            pl.BlockSpec((1, _N, _E), lambda b: (b, 0, 0)),
            pl.BlockSpec((1, 1, _E), lambda b: (b, 0, 0)),
        ),
        out_shape=out_shape,
        compiler_params=pltpu.CompilerParams(
            dimension_semantics=("arbitrary",),
        ),
    )(feat, featT, w0t, w0b, w2t, w2b)
    return Fv2, mean[:, 0, :]
